# baseline (speedup 1.0000x reference)
import functools

import jax
import jax.numpy as jnp
from jax import lax
from jax.experimental import pallas as pl
from jax.experimental.pallas import tpu as pltpu
from jax.experimental.pallas import tpu_sc as plsc

_N = 10000
_D = 128
_H = 128
_LH = 128
_G = 4 * _LH
_E = 320000

_NC = 2
_NS = 16
_NW = _NC * _NS
_CHUNK = 128
_NCHUNKS = _E // _CHUNK
_ITERS = (_NCHUNKS + _NW - 1) // _NW
_ROWS_PER_SUB = 632
_ROWS_LAST = _N - _ROWS_PER_SUB * (_NS - 1)


def _sc_segsum_body(x_hbm, src_hbm, dst_hbm, out_hbm,
                    src_v, dst_v, rows_v, acc_sh):
    cid = lax.axis_index("c")
    sid = lax.axis_index("s")
    wid = sid * _NC + cid

    base = sid * _ROWS_PER_SUB

    @pl.when(sid < _NS - 1)
    def _():
        pltpu.sync_copy(x_hbm.at[pl.ds(base, _ROWS_PER_SUB)],
                        acc_sh.at[pl.ds(base, _ROWS_PER_SUB)])

    @pl.when(sid == _NS - 1)
    def _():
        pltpu.sync_copy(x_hbm.at[pl.ds(base, _ROWS_LAST)],
                        acc_sh.at[pl.ds(base, _ROWS_LAST)])

    plsc.subcore_barrier()

    @pl.loop(0, _ITERS)
    def _(i):
        r = wid + i * _NW

        @pl.when(r < _NCHUNKS)
        def _():
            pltpu.sync_copy(src_hbm.at[r], src_v)
            pltpu.sync_copy(dst_hbm.at[r], dst_v)
            pltpu.sync_copy(x_hbm.at[src_v], rows_v)
            pltpu.sync_copy(rows_v, acc_sh.at[dst_v], add=True)

    plsc.subcore_barrier()

    @pl.when(sid < _NS - 1)
    def _():
        pltpu.sync_copy(acc_sh.at[pl.ds(base, _ROWS_PER_SUB)],
                        out_hbm.at[cid, pl.ds(base, _ROWS_PER_SUB)])

    @pl.when(sid == _NS - 1)
    def _():
        pltpu.sync_copy(acc_sh.at[pl.ds(base, _ROWS_LAST)],
                        out_hbm.at[cid, pl.ds(base, _ROWS_LAST)])


_sc_segsum = functools.partial(
    pl.kernel,
    mesh=plsc.VectorSubcoreMesh(core_axis_name="c", subcore_axis_name="s"),
    out_type=jax.ShapeDtypeStruct((_NC, _N, _D), jnp.float32),
    scratch_types=[
        pltpu.VMEM((_CHUNK,), jnp.int32),
        pltpu.VMEM((_CHUNK,), jnp.int32),
        pltpu.VMEM((_CHUNK, _D), jnp.float32),
        pltpu.VMEM_SHARED((_N, _D), jnp.float32),
    ],
)(_sc_segsum_body)


def _fused_body(x_ref, agg_ref, w1t_ref, b1_ref, w2t_ref, b2_ref,
                gamma_ref, beta_ref, wit_ref, wht_ref, bih_ref,
                wf_ref, bf_ref,
                out_ref, ht_ref, ct_ref, gates_ref, outs_ref):
    h = agg_ref[0, :, :] + agg_ref[1, :, :] - x_ref[...]
    h = jnp.maximum(jnp.dot(h, w1t_ref[...], preferred_element_type=jnp.float32)
                    + b1_ref[...], 0.0)
    h = jnp.dot(h, w2t_ref[...], preferred_element_type=jnp.float32) + b2_ref[...]
    mean = jnp.mean(h, axis=0, keepdims=True)
    var = jnp.mean((h - mean) * (h - mean), axis=0, keepdims=True)
    h = (h - mean) * (gamma_ref[...] * jax.lax.rsqrt(var + 1e-5)) + beta_ref[...]
    h = jnp.maximum(h, 0.0)
    gates_ref[...] = jnp.dot(h, wit_ref[...], preferred_element_type=jnp.float32) \
        + bih_ref[...]

    def step_block(i, carry):
        hp, cp = carry
        base = pl.multiple_of(i * 8, 8)
        gx = gates_ref[pl.ds(base, 8), :]
        rows = []
        for j in range(8):
            hp_col = jnp.transpose(hp)
            g = gx[j:j + 1, :] + jnp.sum(wht_ref[...] * hp_col, axis=0,
                                         keepdims=True)
            ig = jax.nn.sigmoid(g[:, 0:_LH])
            fg = jax.nn.sigmoid(g[:, _LH:2 * _LH])
            gg = jnp.tanh(g[:, 2 * _LH:3 * _LH])
            og = jax.nn.sigmoid(g[:, 3 * _LH:])
            cp = fg * cp + ig * gg
            hp = og * jnp.tanh(cp)
            rows.append(hp)
        outs_ref[pl.ds(base, 8), :] = jnp.concatenate(rows, axis=0)
        return hp, cp

    zero = jnp.zeros((1, _LH), jnp.float32)
    hp, cp = jax.lax.fori_loop(0, _N // 8, step_block, (zero, zero))
    ht_ref[...] = hp
    ct_ref[...] = cp
    out_ref[...] = jnp.sum(outs_ref[...] * wf_ref[...], axis=1, keepdims=True) \
        + bf_ref[...]


@jax.jit
def kernel(x, edge_index, W1, b1, W2, b2, gamma, beta, Wi, Wh, bi, bh, Wf, bf):
    src2d = edge_index[0].reshape(_NCHUNKS, _CHUNK)
    dst2d = edge_index[1].reshape(_NCHUNKS, _CHUNK)
    agg2 = _sc_segsum(x, src2d, dst2d)

    out, hT, cT = pl.pallas_call(
        _fused_body,
        out_shape=(
            jax.ShapeDtypeStruct((_N, 1), jnp.float32),
            jax.ShapeDtypeStruct((1, _LH), jnp.float32),
            jax.ShapeDtypeStruct((1, _LH), jnp.float32),
        ),
        scratch_shapes=[
            pltpu.VMEM((_N, _G), jnp.float32),
            pltpu.VMEM((_N, _LH), jnp.float32),
        ],
        compiler_params=pltpu.CompilerParams(
            vmem_limit_bytes=100 * 1024 * 1024,
        ),
    )(
        x, agg2,
        W1.T, b1[None, :], W2.T, b2[None, :],
        gamma[None, :], beta[None, :],
        Wi.T, Wh.T, (bi + bh)[None, :],
        Wf, bf[None, :],
    )
    return (out.reshape(1, _N), hT[None], cT[None])


# speedup vs baseline: 1.3604x; 1.3604x over previous
import functools

import jax
import jax.numpy as jnp
from jax import lax
from jax.experimental import pallas as pl
from jax.experimental.pallas import tpu as pltpu
from jax.experimental.pallas import tpu_sc as plsc

_N = 10000
_D = 128
_H = 128
_LH = 128
_G = 4 * _LH
_E = 320000

_NC = 2
_NS = 16
_NW = _NC * _NS
_CHUNK = 128
_NCHUNKS = _E // _CHUNK
_ITERS = (_NCHUNKS + _NW - 1) // _NW
_ROWS_PER_SUB = 632
_ROWS_LAST = _N - _ROWS_PER_SUB * (_NS - 1)


def _sc_segsum_body(x_hbm, src_hbm, dst_hbm, out_hbm,
                    src_v, dst_v, rows_v, acc_sh):
    cid = lax.axis_index("c")
    sid = lax.axis_index("s")
    wid = sid * _NC + cid

    base = sid * _ROWS_PER_SUB

    @pl.when(sid < _NS - 1)
    def _():
        pltpu.sync_copy(x_hbm.at[pl.ds(base, _ROWS_PER_SUB)],
                        acc_sh.at[pl.ds(base, _ROWS_PER_SUB)])

    @pl.when(sid == _NS - 1)
    def _():
        pltpu.sync_copy(x_hbm.at[pl.ds(base, _ROWS_LAST)],
                        acc_sh.at[pl.ds(base, _ROWS_LAST)])

    plsc.subcore_barrier()

    @pl.loop(0, _ITERS)
    def _(i):
        r = wid + i * _NW

        @pl.when(r < _NCHUNKS)
        def _():
            pltpu.sync_copy(src_hbm.at[r], src_v)
            pltpu.sync_copy(dst_hbm.at[r], dst_v)
            pltpu.sync_copy(x_hbm.at[src_v], rows_v)
            pltpu.sync_copy(rows_v, acc_sh.at[dst_v], add=True)

    plsc.subcore_barrier()

    @pl.when(sid < _NS - 1)
    def _():
        pltpu.sync_copy(acc_sh.at[pl.ds(base, _ROWS_PER_SUB)],
                        out_hbm.at[cid, pl.ds(base, _ROWS_PER_SUB)])

    @pl.when(sid == _NS - 1)
    def _():
        pltpu.sync_copy(acc_sh.at[pl.ds(base, _ROWS_LAST)],
                        out_hbm.at[cid, pl.ds(base, _ROWS_LAST)])


_sc_segsum = functools.partial(
    pl.kernel,
    mesh=plsc.VectorSubcoreMesh(core_axis_name="c", subcore_axis_name="s"),
    out_type=jax.ShapeDtypeStruct((_NC, _N, _D), jnp.float32),
    scratch_types=[
        pltpu.VMEM((_CHUNK,), jnp.int32),
        pltpu.VMEM((_CHUNK,), jnp.int32),
        pltpu.VMEM((_CHUNK, _D), jnp.float32),
        pltpu.VMEM_SHARED((_N, _D), jnp.float32),
    ],
)(_sc_segsum_body)


def _fused_body(x_ref, agg_ref, w1t_ref, b1_ref, w2t_ref, b2_ref,
                gamma_ref, beta_ref, wit_ref, wht_ref, bih_ref,
                wf_ref, bf_ref,
                out_ref, ht_ref, ct_ref, gates_ref, outs_ref):
    h = agg_ref[0, :, :] + agg_ref[1, :, :] - x_ref[...]
    h = jnp.maximum(jnp.dot(h, w1t_ref[...], preferred_element_type=jnp.float32)
                    + b1_ref[...], 0.0)
    h = jnp.dot(h, w2t_ref[...], preferred_element_type=jnp.float32) + b2_ref[...]
    mean = jnp.mean(h, axis=0, keepdims=True)
    var = jnp.mean((h - mean) * (h - mean), axis=0, keepdims=True)
    h = (h - mean) * (gamma_ref[...] * jax.lax.rsqrt(var + 1e-5)) + beta_ref[...]
    h = jnp.maximum(h, 0.0)
    gates_ref[...] = jnp.dot(h, wit_ref[...], preferred_element_type=jnp.float32) \
        + bih_ref[...]

    wht = wht_ref[...]

    def step_block(i, carry):
        hp, cp = carry
        base = pl.multiple_of(i * 8, 8)
        for j in range(8):
            gx = gates_ref[pl.ds(base + j, 1), :]
            g = gx + jnp.dot(hp, wht, preferred_element_type=jnp.float32)
            ih = jnp.tanh(g[:, 0:_LH] * 0.5)
            fh = jnp.tanh(g[:, _LH:2 * _LH] * 0.5)
            gg = jnp.tanh(g[:, 2 * _LH:3 * _LH])
            oh = jnp.tanh(g[:, 3 * _LH:] * 0.5)
            cp = (0.5 * fh + 0.5) * cp + (0.5 * ih + 0.5) * gg
            hp = (0.5 * oh + 0.5) * jnp.tanh(cp)
            outs_ref[pl.ds(base + j, 1), :] = hp
        return hp, cp

    zero = jnp.zeros((1, _LH), jnp.float32)
    hp, cp = jax.lax.fori_loop(0, _N // 8, step_block, (zero, zero))
    ht_ref[...] = hp
    ct_ref[...] = cp
    out_ref[...] = jnp.sum(outs_ref[...] * wf_ref[...], axis=1, keepdims=True) \
        + bf_ref[...]


@jax.jit
def kernel(x, edge_index, W1, b1, W2, b2, gamma, beta, Wi, Wh, bi, bh, Wf, bf):
    src2d = edge_index[0].reshape(_NCHUNKS, _CHUNK)
    dst2d = edge_index[1].reshape(_NCHUNKS, _CHUNK)
    agg2 = _sc_segsum(x, src2d, dst2d)

    out, hT, cT = pl.pallas_call(
        _fused_body,
        out_shape=(
            jax.ShapeDtypeStruct((_N, 1), jnp.float32),
            jax.ShapeDtypeStruct((1, _LH), jnp.float32),
            jax.ShapeDtypeStruct((1, _LH), jnp.float32),
        ),
        scratch_shapes=[
            pltpu.VMEM((_N, _G), jnp.float32),
            pltpu.VMEM((_N, _LH), jnp.float32),
        ],
        compiler_params=pltpu.CompilerParams(
            vmem_limit_bytes=100 * 1024 * 1024,
        ),
    )(
        x, agg2,
        W1.T, b1[None, :], W2.T, b2[None, :],
        gamma[None, :], beta[None, :],
        Wi.T, Wh.T, (bi + bh)[None, :],
        Wf, bf[None, :],
    )
    return (out.reshape(1, _N), hT[None], cT[None])


# speedup vs baseline: 1.4031x; 1.0314x over previous
import functools

import jax
import jax.numpy as jnp
from jax import lax
from jax.experimental import pallas as pl
from jax.experimental.pallas import tpu as pltpu
from jax.experimental.pallas import tpu_sc as plsc

_N = 10000
_D = 128
_H = 128
_LH = 128
_G = 4 * _LH
_E = 320000

_NC = 2
_NS = 16
_NW = _NC * _NS
_CHUNK = 128
_NCHUNKS = _E // _CHUNK
_BLK = _NCHUNKS // _NW + 1
_PAIRS = (_BLK - 1) // 2
_ROWS_PER_SUB = 632
_ROWS_LAST = _N - _ROWS_PER_SUB * (_NS - 1)


def _sc_segsum_body(x_hbm, src_hbm, dst_hbm, out_hbm,
                    src_v, dst_v, rows0, acc_sh):
    cid = lax.axis_index("c")
    sid = lax.axis_index("s")
    wid = sid * _NC + cid

    base = sid * _ROWS_PER_SUB

    @pl.when(sid < _NS - 1)
    def _():
        pltpu.sync_copy(x_hbm.at[pl.ds(base, _ROWS_PER_SUB)],
                        acc_sh.at[pl.ds(base, _ROWS_PER_SUB)])

    @pl.when(sid == _NS - 1)
    def _():
        pltpu.sync_copy(x_hbm.at[pl.ds(base, _ROWS_LAST)],
                        acc_sh.at[pl.ds(base, _ROWS_LAST)])

    plsc.subcore_barrier()

    pltpu.sync_copy(src_hbm.at[wid], src_v)
    pltpu.sync_copy(dst_hbm.at[wid], dst_v)

    @pl.loop(0, _BLK - 1)
    def _(k):
        pltpu.sync_copy(x_hbm.at[src_v.at[k]], rows0)
        pltpu.sync_copy(rows0, acc_sh.at[dst_v.at[k]], add=True)

    pltpu.sync_copy(x_hbm.at[src_v.at[_BLK - 1]], rows0)

    @pl.when(wid < _NCHUNKS - _NW * (_BLK - 1))
    def _():
        pltpu.sync_copy(rows0, acc_sh.at[dst_v.at[_BLK - 1]], add=True)

    plsc.subcore_barrier()

    @pl.when(sid < _NS - 1)
    def _():
        pltpu.sync_copy(acc_sh.at[pl.ds(base, _ROWS_PER_SUB)],
                        out_hbm.at[cid, pl.ds(base, _ROWS_PER_SUB)])

    @pl.when(sid == _NS - 1)
    def _():
        pltpu.sync_copy(acc_sh.at[pl.ds(base, _ROWS_LAST)],
                        out_hbm.at[cid, pl.ds(base, _ROWS_LAST)])


_sc_segsum = functools.partial(
    pl.kernel,
    mesh=plsc.VectorSubcoreMesh(core_axis_name="c", subcore_axis_name="s"),
    out_type=jax.ShapeDtypeStruct((_NC, _N, _D), jnp.float32),
    scratch_types=[
        pltpu.VMEM((_BLK, _CHUNK), jnp.int32),
        pltpu.VMEM((_BLK, _CHUNK), jnp.int32),
        pltpu.VMEM((_CHUNK, _D), jnp.float32),
        pltpu.VMEM_SHARED((_N, _D), jnp.float32),
    ],
)(_sc_segsum_body)

_EXTRA = _NCHUNKS - _NW * (_BLK - 1)
_ROW_IDS = []
for _w in range(_NW):
    _s = _w * (_BLK - 1) + min(_w, _EXTRA)
    _ROW_IDS.append([_s + _i if _s + _i < _NCHUNKS else _w
                     for _i in range(_BLK)])


def _fused_body(x_ref, agg_ref, w1t_ref, b1_ref, w2t_ref, b2_ref,
                gamma_ref, beta_ref, wit_ref, wht_ref, bih_ref,
                wf_ref, bf_ref,
                out_ref, ht_ref, ct_ref, gates_ref, outs_ref):
    h = agg_ref[0, :, :] + agg_ref[1, :, :] - x_ref[...]
    h = jnp.maximum(jnp.dot(h, w1t_ref[...], preferred_element_type=jnp.float32)
                    + b1_ref[...], 0.0)
    h = jnp.dot(h, w2t_ref[...], preferred_element_type=jnp.float32) + b2_ref[...]
    mean = jnp.mean(h, axis=0, keepdims=True)
    var = jnp.mean((h - mean) * (h - mean), axis=0, keepdims=True)
    h = (h - mean) * (gamma_ref[...] * jax.lax.rsqrt(var + 1e-5)) + beta_ref[...]
    h = jnp.maximum(h, 0.0)
    gates_ref[...] = jnp.dot(h, wit_ref[...], preferred_element_type=jnp.float32) \
        + bih_ref[...]

    wht = wht_ref[...]

    def step_block(i, carry):
        hp, cp = carry
        base = pl.multiple_of(i * 8, 8)
        for j in range(8):
            gx = gates_ref[pl.ds(base + j, 1), :]
            g = gx + jnp.dot(hp, wht, preferred_element_type=jnp.float32)
            ih = jnp.tanh(g[:, 0:_LH] * 0.5)
            fh = jnp.tanh(g[:, _LH:2 * _LH] * 0.5)
            gg = jnp.tanh(g[:, 2 * _LH:3 * _LH])
            oh = jnp.tanh(g[:, 3 * _LH:] * 0.5)
            cp = (0.5 * fh + 0.5) * cp + (0.5 * ih + 0.5) * gg
            hp = (0.5 * oh + 0.5) * jnp.tanh(cp)
            outs_ref[pl.ds(base + j, 1), :] = hp
        return hp, cp

    zero = jnp.zeros((1, _LH), jnp.float32)
    hp, cp = jax.lax.fori_loop(0, _N // 8, step_block, (zero, zero))
    ht_ref[...] = hp
    ct_ref[...] = cp
    out_ref[...] = jnp.sum(outs_ref[...] * wf_ref[...], axis=1, keepdims=True) \
        + bf_ref[...]


@jax.jit
def kernel(x, edge_index, W1, b1, W2, b2, gamma, beta, Wi, Wh, bi, bh, Wf, bf):
    row_ids = jnp.asarray(_ROW_IDS, dtype=jnp.int32)
    src3d = jnp.take(edge_index[0].reshape(_NCHUNKS, _CHUNK), row_ids, axis=0)
    dst3d = jnp.take(edge_index[1].reshape(_NCHUNKS, _CHUNK), row_ids, axis=0)
    agg2 = _sc_segsum(x, src3d, dst3d)

    out, hT, cT = pl.pallas_call(
        _fused_body,
        out_shape=(
            jax.ShapeDtypeStruct((_N, 1), jnp.float32),
            jax.ShapeDtypeStruct((1, _LH), jnp.float32),
            jax.ShapeDtypeStruct((1, _LH), jnp.float32),
        ),
        scratch_shapes=[
            pltpu.VMEM((_N, _G), jnp.float32),
            pltpu.VMEM((_N, _LH), jnp.float32),
        ],
        compiler_params=pltpu.CompilerParams(
            vmem_limit_bytes=100 * 1024 * 1024,
        ),
    )(
        x, agg2,
        W1.T, b1[None, :], W2.T, b2[None, :],
        gamma[None, :], beta[None, :],
        Wi.T, Wh.T, (bi + bh)[None, :],
        Wf, bf[None, :],
    )
    return (out.reshape(1, _N), hT[None], cT[None])


# speedup vs baseline: 1.4132x; 1.0072x over previous
import functools

import numpy as np
import jax
import jax.numpy as jnp
from jax import lax
from jax.experimental import pallas as pl
from jax.experimental.pallas import tpu as pltpu
from jax.experimental.pallas import tpu_sc as plsc

_N = 10000
_D = 128
_H = 128
_LH = 128
_G = 4 * _LH
_E = 320000

_NC = 2
_NS = 16
_NW = _NC * _NS
_CHUNK = 128
_NCHUNKS = _E // _CHUNK
_BLK = _NCHUNKS // _NW + 1
_PAIRS = (_BLK - 1) // 2
_ROWS_PER_SUB = 632
_ROWS_LAST = _N - _ROWS_PER_SUB * (_NS - 1)


def _sc_segsum_body(x_hbm, src_hbm, dst_hbm, out_hbm,
                    src_v, dst_v, rows0, acc_sh):
    cid = lax.axis_index("c")
    sid = lax.axis_index("s")
    wid = sid * _NC + cid

    base = sid * _ROWS_PER_SUB

    @pl.when(sid < _NS - 1)
    def _():
        pltpu.sync_copy(x_hbm.at[pl.ds(base, _ROWS_PER_SUB)],
                        acc_sh.at[pl.ds(base, _ROWS_PER_SUB)])

    @pl.when(sid == _NS - 1)
    def _():
        pltpu.sync_copy(x_hbm.at[pl.ds(base, _ROWS_LAST)],
                        acc_sh.at[pl.ds(base, _ROWS_LAST)])

    plsc.subcore_barrier()

    pltpu.sync_copy(src_hbm.at[wid], src_v)
    pltpu.sync_copy(dst_hbm.at[wid], dst_v)

    @pl.loop(0, _BLK - 1)
    def _(k):
        pltpu.sync_copy(x_hbm.at[src_v.at[k]], rows0)
        pltpu.sync_copy(rows0, acc_sh.at[dst_v.at[k]], add=True)

    pltpu.sync_copy(x_hbm.at[src_v.at[_BLK - 1]], rows0)

    @pl.when(wid < _NCHUNKS - _NW * (_BLK - 1))
    def _():
        pltpu.sync_copy(rows0, acc_sh.at[dst_v.at[_BLK - 1]], add=True)

    plsc.subcore_barrier()

    @pl.when(sid < _NS - 1)
    def _():
        pltpu.sync_copy(acc_sh.at[pl.ds(base, _ROWS_PER_SUB)],
                        out_hbm.at[cid, pl.ds(base, _ROWS_PER_SUB)])

    @pl.when(sid == _NS - 1)
    def _():
        pltpu.sync_copy(acc_sh.at[pl.ds(base, _ROWS_LAST)],
                        out_hbm.at[cid, pl.ds(base, _ROWS_LAST)])


_sc_segsum = functools.partial(
    pl.kernel,
    mesh=plsc.VectorSubcoreMesh(core_axis_name="c", subcore_axis_name="s"),
    out_type=jax.ShapeDtypeStruct((_NC, _N, _D), jnp.float32),
    scratch_types=[
        pltpu.VMEM((_BLK, _CHUNK), jnp.int32),
        pltpu.VMEM((_BLK, _CHUNK), jnp.int32),
        pltpu.VMEM((_CHUNK, _D), jnp.float32),
        pltpu.VMEM_SHARED((_N, _D), jnp.float32),
    ],
)(_sc_segsum_body)

_EXTRA = _NCHUNKS - _NW * (_BLK - 1)
_ROW_IDS = []
for _w in range(_NW):
    _s = _w * (_BLK - 1) + min(_w, _EXTRA)
    _ROW_IDS.append([_s + _i if _s + _i < _NCHUNKS else _w
                     for _i in range(_BLK)])


def _fused_body(x_ref, agg_ref, w1t_ref, b1_ref, w2t_ref, b2_ref,
                gamma_ref, beta_ref, wit_ref, wht_ref, bih_ref,
                wf_ref, bf_ref,
                out_ref, ht_ref, ct_ref, gates_ref, outs_ref):
    h = agg_ref[0, :, :] + agg_ref[1, :, :] - x_ref[...]
    h = jnp.maximum(jnp.dot(h, w1t_ref[...], preferred_element_type=jnp.float32)
                    + b1_ref[...], 0.0)
    h = jnp.dot(h, w2t_ref[...], preferred_element_type=jnp.float32) + b2_ref[...]
    mean = jnp.mean(h, axis=0, keepdims=True)
    var = jnp.mean((h - mean) * (h - mean), axis=0, keepdims=True)
    h = (h - mean) * (gamma_ref[...] * jax.lax.rsqrt(var + 1e-5)) + beta_ref[...]
    h = jnp.maximum(h, 0.0)
    gates_ref[...] = jnp.dot(h, wit_ref[...], preferred_element_type=jnp.float32) \
        + bih_ref[...]

    wht = wht_ref[...]

    def step_block(i, carry):
        hp, cp = carry
        base = pl.multiple_of(i * 8, 8)
        for j in range(8):
            gx = gates_ref[pl.ds(base + j, 1), :]
            g = gx + jnp.dot(hp, wht, preferred_element_type=jnp.float32)
            ih = jnp.tanh(g[:, 0:_LH])
            fh = jnp.tanh(g[:, _LH:2 * _LH])
            gg = jnp.tanh(g[:, 2 * _LH:3 * _LH])
            oh = jnp.tanh(g[:, 3 * _LH:])
            cp = (0.5 * fh + 0.5) * cp + (0.5 * ih + 0.5) * gg
            hp = (0.5 * oh + 0.5) * jnp.tanh(cp)
            outs_ref[pl.ds(base + j, 1), :] = hp
        return hp, cp

    zero = jnp.zeros((1, _LH), jnp.float32)
    hp, cp = jax.lax.fori_loop(0, _N // 8, step_block, (zero, zero))
    ht_ref[...] = hp
    ct_ref[...] = cp
    out_ref[...] = jnp.sum(outs_ref[...] * wf_ref[...], axis=1, keepdims=True) \
        + bf_ref[...]


_GSCALE = np.concatenate(
    [np.full(_LH, 0.5), np.full(_LH, 0.5),
     np.ones(_LH), np.full(_LH, 0.5)]).astype(np.float32)[None, :]


@jax.jit
def kernel(x, edge_index, W1, b1, W2, b2, gamma, beta, Wi, Wh, bi, bh, Wf, bf):
    row_ids = jnp.asarray(_ROW_IDS, dtype=jnp.int32)
    src3d = jnp.take(edge_index[0].reshape(_NCHUNKS, _CHUNK), row_ids, axis=0)
    dst3d = jnp.take(edge_index[1].reshape(_NCHUNKS, _CHUNK), row_ids, axis=0)
    agg2 = _sc_segsum(x, src3d, dst3d)

    out, hT, cT = pl.pallas_call(
        _fused_body,
        out_shape=(
            jax.ShapeDtypeStruct((_N, 1), jnp.float32),
            jax.ShapeDtypeStruct((1, _LH), jnp.float32),
            jax.ShapeDtypeStruct((1, _LH), jnp.float32),
        ),
        scratch_shapes=[
            pltpu.VMEM((_N, _G), jnp.float32),
            pltpu.VMEM((_N, _LH), jnp.float32),
        ],
        compiler_params=pltpu.CompilerParams(
            vmem_limit_bytes=100 * 1024 * 1024,
        ),
    )(
        x, agg2,
        W1.T, b1[None, :], W2.T, b2[None, :],
        gamma[None, :], beta[None, :],
        Wi.T * _GSCALE, Wh.T * _GSCALE, ((bi + bh) * _GSCALE[0])[None, :],
        Wf, bf[None, :],
    )
    return (out.reshape(1, _N), hT[None], cT[None])


# speedup vs baseline: 1.4615x; 1.0341x over previous
import functools

import numpy as np
import jax
import jax.numpy as jnp
from jax import lax
from jax.experimental import pallas as pl
from jax.experimental.pallas import tpu as pltpu
from jax.experimental.pallas import tpu_sc as plsc

_N = 10000
_D = 128
_H = 128
_LH = 128
_G = 4 * _LH
_E = 320000

_NC = 2
_NS = 16
_NW = _NC * _NS
_CHUNK = 128
_NCHUNKS = _E // _CHUNK
_BLK = _NCHUNKS // _NW + 1
_PAIRS = (_BLK - 1) // 2
_ROWS_PER_SUB = 632
_ROWS_LAST = _N - _ROWS_PER_SUB * (_NS - 1)


def _sc_segsum_body(x_hbm, src_hbm, dst_hbm, out_hbm,
                    src0, src1, dst_v, rows0, rows1, acc_sh, sem0, sem1):
    cid = lax.axis_index("c")
    sid = lax.axis_index("s")
    wid = sid * _NC + cid

    base = sid * _ROWS_PER_SUB

    @pl.when(sid < _NS - 1)
    def _():
        pltpu.sync_copy(x_hbm.at[pl.ds(base, _ROWS_PER_SUB)],
                        acc_sh.at[pl.ds(base, _ROWS_PER_SUB)])

    @pl.when(sid == _NS - 1)
    def _():
        pltpu.sync_copy(x_hbm.at[pl.ds(base, _ROWS_LAST)],
                        acc_sh.at[pl.ds(base, _ROWS_LAST)])

    plsc.subcore_barrier()

    pltpu.sync_copy(dst_hbm.at[wid], dst_v)
    pltpu.sync_copy(src_hbm.at[wid, 0], src0)
    pltpu.sync_copy(src_hbm.at[wid, 1], src1)
    pltpu.async_copy(x_hbm.at[src0], rows0, sem0)
    pltpu.async_copy(x_hbm.at[src1], rows1, sem1)

    @pl.loop(0, _PAIRS)
    def _(i):
        k0 = 2 * i

        pltpu.make_async_copy(x_hbm.at[src0], rows0, sem0).wait()
        pltpu.sync_copy(rows0, acc_sh.at[dst_v.at[k0]], add=True)

        @pl.when(k0 + 2 < _BLK)
        def _():
            pltpu.sync_copy(src_hbm.at[wid, k0 + 2], src0)
            pltpu.async_copy(x_hbm.at[src0], rows0, sem0)

        pltpu.make_async_copy(x_hbm.at[src1], rows1, sem1).wait()
        pltpu.sync_copy(rows1, acc_sh.at[dst_v.at[k0 + 1]], add=True)

        @pl.when(k0 + 3 < _BLK)
        def _():
            pltpu.sync_copy(src_hbm.at[wid, k0 + 3], src1)
            pltpu.async_copy(x_hbm.at[src1], rows1, sem1)

    pltpu.make_async_copy(x_hbm.at[src0], rows0, sem0).wait()

    @pl.when(wid < _NCHUNKS - _NW * (_BLK - 1))
    def _():
        pltpu.sync_copy(rows0, acc_sh.at[dst_v.at[_BLK - 1]], add=True)

    plsc.subcore_barrier()

    @pl.when(sid < _NS - 1)
    def _():
        pltpu.sync_copy(acc_sh.at[pl.ds(base, _ROWS_PER_SUB)],
                        out_hbm.at[cid, pl.ds(base, _ROWS_PER_SUB)])

    @pl.when(sid == _NS - 1)
    def _():
        pltpu.sync_copy(acc_sh.at[pl.ds(base, _ROWS_LAST)],
                        out_hbm.at[cid, pl.ds(base, _ROWS_LAST)])


_sc_segsum = functools.partial(
    pl.kernel,
    mesh=plsc.VectorSubcoreMesh(core_axis_name="c", subcore_axis_name="s"),
    out_type=jax.ShapeDtypeStruct((_NC, _N, _D), jnp.float32),
    scratch_types=[
        pltpu.VMEM((_CHUNK,), jnp.int32),
        pltpu.VMEM((_CHUNK,), jnp.int32),
        pltpu.VMEM((_BLK, _CHUNK), jnp.int32),
        pltpu.VMEM((_CHUNK, _D), jnp.float32),
        pltpu.VMEM((_CHUNK, _D), jnp.float32),
        pltpu.VMEM_SHARED((_N, _D), jnp.float32),
        pltpu.SemaphoreType.DMA,
        pltpu.SemaphoreType.DMA,
    ],
)(_sc_segsum_body)

_EXTRA = _NCHUNKS - _NW * (_BLK - 1)
_ROW_IDS = []
for _w in range(_NW):
    _s = _w * (_BLK - 1) + min(_w, _EXTRA)
    _ROW_IDS.append([_s + _i if _s + _i < _NCHUNKS else _w
                     for _i in range(_BLK)])


def _fused_body(x_ref, agg_ref, w1t_ref, b1_ref, w2t_ref, b2_ref,
                gamma_ref, beta_ref, wit_ref, wht_ref, bih_ref,
                wf_ref, bf_ref,
                out_ref, ht_ref, ct_ref, gates_ref, outs_ref):
    h = agg_ref[0, :, :] + agg_ref[1, :, :] - x_ref[...]
    h = jnp.maximum(jnp.dot(h, w1t_ref[...], preferred_element_type=jnp.float32)
                    + b1_ref[...], 0.0)
    h = jnp.dot(h, w2t_ref[...], preferred_element_type=jnp.float32) + b2_ref[...]
    mean = jnp.mean(h, axis=0, keepdims=True)
    var = jnp.mean((h - mean) * (h - mean), axis=0, keepdims=True)
    h = (h - mean) * (gamma_ref[...] * jax.lax.rsqrt(var + 1e-5)) + beta_ref[...]
    h = jnp.maximum(h, 0.0)
    gates_ref[...] = jnp.dot(h, wit_ref[...], preferred_element_type=jnp.float32) \
        + bih_ref[...]

    wht = wht_ref[...]

    def step_block(i, carry):
        hp, cp = carry
        base = pl.multiple_of(i * 8, 8)
        for j in range(8):
            gx = gates_ref[pl.ds(base + j, 1), :]
            g = gx + jnp.dot(hp, wht, preferred_element_type=jnp.float32)
            ih = jnp.tanh(g[:, 0:_LH])
            fh = jnp.tanh(g[:, _LH:2 * _LH])
            gg = jnp.tanh(g[:, 2 * _LH:3 * _LH])
            oh = jnp.tanh(g[:, 3 * _LH:])
            cp = (0.5 * fh + 0.5) * cp + (0.5 * ih + 0.5) * gg
            hp = (0.5 * oh + 0.5) * jnp.tanh(cp)
            outs_ref[pl.ds(base + j, 1), :] = hp
        return hp, cp

    zero = jnp.zeros((1, _LH), jnp.float32)
    hp, cp = jax.lax.fori_loop(0, _N // 8, step_block, (zero, zero))
    ht_ref[...] = hp
    ct_ref[...] = cp
    out_ref[...] = jnp.sum(outs_ref[...] * wf_ref[...], axis=1, keepdims=True) \
        + bf_ref[...]


_GSCALE = np.concatenate(
    [np.full(_LH, 0.5), np.full(_LH, 0.5),
     np.ones(_LH), np.full(_LH, 0.5)]).astype(np.float32)[None, :]


@jax.jit
def kernel(x, edge_index, W1, b1, W2, b2, gamma, beta, Wi, Wh, bi, bh, Wf, bf):
    row_ids = jnp.asarray(_ROW_IDS, dtype=jnp.int32)
    src3d = jnp.take(edge_index[0].reshape(_NCHUNKS, _CHUNK), row_ids, axis=0)
    dst3d = jnp.take(edge_index[1].reshape(_NCHUNKS, _CHUNK), row_ids, axis=0)
    agg2 = _sc_segsum(x, src3d, dst3d)

    out, hT, cT = pl.pallas_call(
        _fused_body,
        out_shape=(
            jax.ShapeDtypeStruct((_N, 1), jnp.float32),
            jax.ShapeDtypeStruct((1, _LH), jnp.float32),
            jax.ShapeDtypeStruct((1, _LH), jnp.float32),
        ),
        scratch_shapes=[
            pltpu.VMEM((_N, _G), jnp.float32),
            pltpu.VMEM((_N, _LH), jnp.float32),
        ],
        compiler_params=pltpu.CompilerParams(
            vmem_limit_bytes=100 * 1024 * 1024,
        ),
    )(
        x, agg2,
        W1.T, b1[None, :], W2.T, b2[None, :],
        gamma[None, :], beta[None, :],
        Wi.T * _GSCALE, Wh.T * _GSCALE, ((bi + bh) * _GSCALE[0])[None, :],
        Wf, bf[None, :],
    )
    return (out.reshape(1, _N), hT[None], cT[None])
